# baseline (speedup 1.0000x reference)
import functools

import jax
import jax.numpy as jnp
from jax import lax
from jax.experimental import pallas as pl
from jax.experimental.pallas import tpu as pltpu
from jax.experimental.pallas import tpu_sc as plsc

VOCAB = 100000
EMBED_DIM = 128
HIDDEN_DIM = 256
NUM_CLASSES = 2
BATCH = 4096
SEQ_LEN = 200

NUM_CORES = 2
NUM_SUBCORES = 16
NUM_WORKERS = NUM_CORES * NUM_SUBCORES
BPW = BATCH // NUM_WORKERS
LANES = 16
PW = EMBED_DIM // 2
PCH = PW // LANES


def _reduce_and_store(buf_v, bi, out_v, samp):
    init = tuple(jnp.zeros((2 * LANES,), jnp.bfloat16) for _ in range(2 * PCH))

    @plsc.parallel_loop(0, SEQ_LEN // 2, unroll=4, carry=init)
    def accs(r2, accs):
        r = r2 * 2
        a_new = [
            accs[d] + plsc.bitcast(buf_v[bi, r, pl.ds(d * LANES, LANES)], jnp.bfloat16)
            for d in range(PCH)
        ]
        b_new = [
            accs[PCH + d]
            + plsc.bitcast(buf_v[bi, r + 1, pl.ds(d * LANES, LANES)], jnp.bfloat16)
            for d in range(PCH)
        ]
        return tuple(a_new + b_new)
    for d in range(PCH):
        out_v[samp, pl.ds(d * LANES, LANES)] = plsc.bitcast(
            accs[d] + accs[PCH + d], jnp.int32)


_sc_mesh = plsc.VectorSubcoreMesh(core_axis_name="c", subcore_axis_name="s")


def _make_gather_sum(batch):
    bpw = batch // NUM_WORKERS

    @functools.partial(
        pl.kernel,
        out_type=jax.ShapeDtypeStruct((batch, PW), jnp.int32),
        mesh=_sc_mesh,
        scratch_types=[
            pltpu.VMEM((bpw * SEQ_LEN,), jnp.int32),
            pltpu.VMEM((2, SEQ_LEN, PW), jnp.int32),
            pltpu.VMEM((bpw, PW), jnp.int32),
            pltpu.SemaphoreType.DMA,
            pltpu.SemaphoreType.DMA,
        ],
        compiler_params=pltpu.CompilerParams(
            use_tc_tiling_on_sc=False, needs_layout_passes=False),
    )
    def _sc_gather_sum(idx_hbm, table_hbm, out_hbm, idx_v, buf_v, out_v, sem0, sem1):
        wid = lax.axis_index("s") * NUM_CORES + lax.axis_index("c")
        base = wid * bpw
        pltpu.sync_copy(idx_hbm.at[pl.ds(base * SEQ_LEN, bpw * SEQ_LEN)], idx_v)
        pltpu.async_copy(table_hbm.at[idx_v.at[pl.ds(0, SEQ_LEN)]], buf_v.at[0], sem0)

        def two_samples(i, carry):
            s0 = 2 * i
            pltpu.async_copy(
                table_hbm.at[idx_v.at[pl.ds((s0 + 1) * SEQ_LEN, SEQ_LEN)]], buf_v.at[1], sem1)
            pltpu.make_async_copy(
                table_hbm.at[pl.ds(0, SEQ_LEN)], buf_v.at[0], sem0).wait()
            _reduce_and_store(buf_v, 0, out_v, s0)

            @pl.when(s0 + 2 < bpw)
            def _():
                pltpu.async_copy(
                    table_hbm.at[idx_v.at[pl.ds((s0 + 2) * SEQ_LEN, SEQ_LEN)]],
                    buf_v.at[0], sem0)

            pltpu.make_async_copy(
                table_hbm.at[pl.ds(0, SEQ_LEN)], buf_v.at[1], sem1).wait()
            _reduce_and_store(buf_v, 1, out_v, s0 + 1)
            return carry

        lax.fori_loop(0, bpw // 2, two_samples, 0)
        pltpu.sync_copy(out_v, out_hbm.at[pl.ds(base, bpw)])

    return _sc_gather_sum


_BATCH_H = BATCH // 2
_sc_gather_sum_half = _make_gather_sum(_BATCH_H)


_BB = 1024


def _mlp_body(x_ref, w1_ref, b1_ref, w2_ref, b2_ref, o_ref):
    w = x_ref[...]
    lo = lax.bitcast_convert_type(w << 16, jnp.float32)
    hi = lax.bitcast_convert_type(w & jnp.int32(-65536), jnp.float32)
    x = jnp.concatenate([lo, hi], axis=1) * jnp.float32(1.0 / SEQ_LEN)
    h = jnp.dot(x, w1_ref[...], preferred_element_type=jnp.float32)
    h = jnp.maximum(h + b1_ref[...], 0.0)
    logits = jnp.dot(h, w2_ref[...], preferred_element_type=jnp.float32)
    logits = logits + b2_ref[...]
    m = jnp.max(logits, axis=1, keepdims=True)
    lse = m + jnp.log(jnp.sum(jnp.exp(logits - m), axis=1, keepdims=True))
    o_ref[...] = (logits - lse)[:, :NUM_CLASSES]


def _tc_mlp(x, w1, b1, w2p, b2p):
    batch = x.shape[0]
    grid = (batch // _BB,)
    return pl.pallas_call(
        _mlp_body,
        grid=grid,
        in_specs=[
            pl.BlockSpec((_BB, PW), lambda i: (i, 0)),
            pl.BlockSpec((EMBED_DIM, HIDDEN_DIM), lambda i: (0, 0)),
            pl.BlockSpec((1, HIDDEN_DIM), lambda i: (0, 0)),
            pl.BlockSpec((HIDDEN_DIM, EMBED_DIM), lambda i: (0, 0)),
            pl.BlockSpec((1, EMBED_DIM), lambda i: (0, 0)),
        ],
        out_specs=pl.BlockSpec((_BB, NUM_CLASSES), lambda i: (i, 0)),
        out_shape=jax.ShapeDtypeStruct((batch, NUM_CLASSES), jnp.float32),
    )(x, w1, b1, w2p, b2p)


ROWS_PER_WORKER = VOCAB // NUM_WORKERS
PACK_CHUNK = 300
_PACK_SPANS = [(c * PACK_CHUNK, PACK_CHUNK) for c in range(10)] + [(3000, 125)]


@functools.partial(
    pl.kernel,
    out_type=jax.ShapeDtypeStruct((VOCAB, PW), jnp.int32),
    mesh=_sc_mesh,
    scratch_types=[
        pltpu.VMEM((2, PACK_CHUNK, EMBED_DIM), jnp.float32),
        pltpu.VMEM((2, PACK_CHUNK, PW), jnp.int32),
        pltpu.SemaphoreType.DMA,
        pltpu.SemaphoreType.DMA,
    ],
    compiler_params=pltpu.CompilerParams(
        use_tc_tiling_on_sc=False, needs_layout_passes=False),
)
def _sc_pack_table(table_hbm, out_hbm, in_v, out_v, sem_in, sem_out):
    wid = lax.axis_index("s") * NUM_CORES + lax.axis_index("c")
    rbase = wid * ROWS_PER_WORKER
    n_chunks = len(_PACK_SPANS)
    pltpu.async_copy(
        table_hbm.at[pl.ds(rbase, _PACK_SPANS[0][1])],
        in_v.at[0, pl.ds(0, _PACK_SPANS[0][1])], sem_in)

    def pack_chunk(p, nrows):
        @plsc.parallel_loop(0, nrows, unroll=5)
        def row_body(r):
            for j in range(PCH):
                a = in_v[p, r, pl.ds(j * LANES, LANES)]
                b = in_v[p, r, pl.ds(PW + j * LANES, LANES)]
                out_v[p, r, pl.ds(j * LANES, LANES)] = plsc.bitcast(
                    plsc.pack(a, b, format=plsc.PackFormat.INTERLEAVED), jnp.int32)

    for c, (off, nrows) in enumerate(_PACK_SPANS):
        p = c % 2
        pltpu.make_async_copy(
            table_hbm.at[pl.ds(rbase + off, nrows)],
            in_v.at[p, pl.ds(0, nrows)], sem_in).wait()
        if c + 1 < n_chunks:
            noff, nn = _PACK_SPANS[c + 1]
            pltpu.async_copy(
                table_hbm.at[pl.ds(rbase + noff, nn)],
                in_v.at[1 - p, pl.ds(0, nn)], sem_in)
        if c >= 2:
            poff, pn = _PACK_SPANS[c - 2]
            pltpu.make_async_copy(
                out_v.at[p, pl.ds(0, pn)],
                out_hbm.at[pl.ds(rbase + poff, pn)], sem_out).wait()
        pack_chunk(p, nrows)
        pltpu.async_copy(
            out_v.at[p, pl.ds(0, nrows)],
            out_hbm.at[pl.ds(rbase + off, nrows)], sem_out)
    for c in (n_chunks - 2, n_chunks - 1):
        off, nrows = _PACK_SPANS[c]
        pltpu.make_async_copy(
            out_v.at[c % 2, pl.ds(0, nrows)],
            out_hbm.at[pl.ds(rbase + off, nrows)], sem_out).wait()


def kernel(input_batch, embedding_table, W1, b1, W2, b2):
    idx = input_batch.astype(jnp.int32).reshape(BATCH * SEQ_LEN)
    packed = _sc_pack_table(embedding_table)
    sums0 = _sc_gather_sum_half(idx[: _BATCH_H * SEQ_LEN], packed)
    sums1 = _sc_gather_sum_half(idx[_BATCH_H * SEQ_LEN:], packed)
    w2p = jnp.pad(W2, ((0, 0), (0, EMBED_DIM - NUM_CLASSES)))
    b2p = jnp.pad(
        b2.reshape(1, NUM_CLASSES),
        ((0, 0), (0, EMBED_DIM - NUM_CLASSES)),
        constant_values=-1e30,
    )
    b1r = b1.reshape(1, HIDDEN_DIM)
    out0 = _tc_mlp(sums0, W1, b1r, w2p, b2p)
    out1 = _tc_mlp(sums1, W1, b1r, w2p, b2p)
    return jnp.concatenate([out0, out1], axis=0)


# speedup vs baseline: 1.0328x; 1.0328x over previous
import functools

import jax
import jax.numpy as jnp
from jax import lax
from jax.experimental import pallas as pl
from jax.experimental.pallas import tpu as pltpu
from jax.experimental.pallas import tpu_sc as plsc

VOCAB = 100000
EMBED_DIM = 128
HIDDEN_DIM = 256
NUM_CLASSES = 2
BATCH = 4096
SEQ_LEN = 200

NUM_CORES = 2
NUM_SUBCORES = 16
NUM_WORKERS = NUM_CORES * NUM_SUBCORES
BPW = BATCH // NUM_WORKERS
LANES = 16
PW = EMBED_DIM // 2
PCH = PW // LANES


def _reduce_and_store(buf_v, bi, out_v, samp):
    init = tuple(jnp.zeros((2 * LANES,), jnp.bfloat16) for _ in range(2 * PCH))

    @plsc.parallel_loop(0, SEQ_LEN // 2, unroll=4, carry=init)
    def accs(r2, accs):
        r = r2 * 2
        a_new = [
            accs[d] + plsc.bitcast(buf_v[bi, r, pl.ds(d * LANES, LANES)], jnp.bfloat16)
            for d in range(PCH)
        ]
        b_new = [
            accs[PCH + d]
            + plsc.bitcast(buf_v[bi, r + 1, pl.ds(d * LANES, LANES)], jnp.bfloat16)
            for d in range(PCH)
        ]
        return tuple(a_new + b_new)
    for d in range(PCH):
        out_v[samp, pl.ds(d * LANES, LANES)] = plsc.bitcast(
            accs[d] + accs[PCH + d], jnp.int32)


_sc_mesh = plsc.VectorSubcoreMesh(core_axis_name="c", subcore_axis_name="s")


@functools.partial(
    pl.kernel,
    out_type=jax.ShapeDtypeStruct((BATCH, PW), jnp.int32),
    mesh=_sc_mesh,
    scratch_types=[
        pltpu.VMEM((BPW * SEQ_LEN,), jnp.int32),
        pltpu.VMEM((2, SEQ_LEN, PW), jnp.int32),
        pltpu.VMEM((BPW, PW), jnp.int32),
        pltpu.SemaphoreType.DMA,
        pltpu.SemaphoreType.DMA,
    ],
    compiler_params=pltpu.CompilerParams(
        use_tc_tiling_on_sc=False, needs_layout_passes=False),
)
def _sc_gather_sum(idx_hbm, table_hbm, out_hbm, idx_v, buf_v, out_v, sem0, sem1):
    wid = lax.axis_index("s") * NUM_CORES + lax.axis_index("c")
    base = wid * BPW
    pltpu.sync_copy(idx_hbm.at[pl.ds(base * SEQ_LEN, BPW * SEQ_LEN)], idx_v)
    pltpu.async_copy(table_hbm.at[idx_v.at[pl.ds(0, SEQ_LEN)]], buf_v.at[0], sem0)

    def two_samples(i, carry):
        s0 = 2 * i
        pltpu.async_copy(
            table_hbm.at[idx_v.at[pl.ds((s0 + 1) * SEQ_LEN, SEQ_LEN)]], buf_v.at[1], sem1)
        pltpu.make_async_copy(table_hbm.at[pl.ds(0, SEQ_LEN)], buf_v.at[0], sem0).wait()
        _reduce_and_store(buf_v, 0, out_v, s0)

        @pl.when(s0 + 2 < BPW)
        def _():
            pltpu.async_copy(
                table_hbm.at[idx_v.at[pl.ds((s0 + 2) * SEQ_LEN, SEQ_LEN)]], buf_v.at[0], sem0)

        pltpu.make_async_copy(table_hbm.at[pl.ds(0, SEQ_LEN)], buf_v.at[1], sem1).wait()
        _reduce_and_store(buf_v, 1, out_v, s0 + 1)
        return carry

    lax.fori_loop(0, BPW // 2, two_samples, 0)
    pltpu.sync_copy(out_v, out_hbm.at[pl.ds(base, BPW)])


_BB = 1024


def _mlp_body(x_ref, w1_ref, b1_ref, w2_ref, b2_ref, o_ref):
    w = x_ref[...]
    lo = lax.bitcast_convert_type(w << 16, jnp.float32)
    hi = lax.bitcast_convert_type(w & jnp.int32(-65536), jnp.float32)
    x = jnp.concatenate([lo, hi], axis=1) * jnp.float32(1.0 / SEQ_LEN)
    h = jnp.dot(x, w1_ref[...], preferred_element_type=jnp.float32)
    h = jnp.maximum(h + b1_ref[...], 0.0)
    logits = jnp.dot(h, w2_ref[...], preferred_element_type=jnp.float32)
    logits = logits + b2_ref[...]
    m = jnp.max(logits, axis=1, keepdims=True)
    lse = m + jnp.log(jnp.sum(jnp.exp(logits - m), axis=1, keepdims=True))
    o_ref[...] = (logits - lse)[:, :NUM_CLASSES]


def _tc_mlp(x, w1, b1, w2p, b2p):
    grid = (BATCH // _BB,)
    return pl.pallas_call(
        _mlp_body,
        grid=grid,
        in_specs=[
            pl.BlockSpec((_BB, PW), lambda i: (i, 0)),
            pl.BlockSpec((EMBED_DIM, HIDDEN_DIM), lambda i: (0, 0)),
            pl.BlockSpec((1, HIDDEN_DIM), lambda i: (0, 0)),
            pl.BlockSpec((HIDDEN_DIM, EMBED_DIM), lambda i: (0, 0)),
            pl.BlockSpec((1, EMBED_DIM), lambda i: (0, 0)),
        ],
        out_specs=pl.BlockSpec((_BB, NUM_CLASSES), lambda i: (i, 0)),
        out_shape=jax.ShapeDtypeStruct((BATCH, NUM_CLASSES), jnp.float32),
    )(x, w1, b1, w2p, b2p)


ROWS_PER_WORKER = VOCAB // NUM_WORKERS
PACK_CHUNK = 300
_PACK_SPANS = [(c * PACK_CHUNK, PACK_CHUNK) for c in range(10)] + [(3000, 125)]


@functools.partial(
    pl.kernel,
    out_type=jax.ShapeDtypeStruct((VOCAB, PW), jnp.int32),
    mesh=_sc_mesh,
    scratch_types=[
        pltpu.VMEM((2, PACK_CHUNK, EMBED_DIM), jnp.float32),
        pltpu.VMEM((2, PACK_CHUNK, PW), jnp.int32),
        pltpu.SemaphoreType.DMA,
        pltpu.SemaphoreType.DMA,
    ],
    compiler_params=pltpu.CompilerParams(
        use_tc_tiling_on_sc=False, needs_layout_passes=False),
)
def _sc_pack_table(table_hbm, out_hbm, in_v, out_v, sem_in, sem_out):
    wid = lax.axis_index("s") * NUM_CORES + lax.axis_index("c")
    rbase = wid * ROWS_PER_WORKER
    n_chunks = len(_PACK_SPANS)
    pltpu.async_copy(
        table_hbm.at[pl.ds(rbase, _PACK_SPANS[0][1])],
        in_v.at[0, pl.ds(0, _PACK_SPANS[0][1])], sem_in)

    def pack_chunk(p, nrows):
        @plsc.parallel_loop(0, nrows, unroll=5)
        def row_body(r):
            for j in range(PCH):
                a = in_v[p, r, pl.ds(j * LANES, LANES)]
                b = in_v[p, r, pl.ds(PW + j * LANES, LANES)]
                out_v[p, r, pl.ds(j * LANES, LANES)] = plsc.bitcast(
                    plsc.pack(a, b, format=plsc.PackFormat.INTERLEAVED), jnp.int32)

    for c, (off, nrows) in enumerate(_PACK_SPANS):
        p = c % 2
        pltpu.make_async_copy(
            table_hbm.at[pl.ds(rbase + off, nrows)],
            in_v.at[p, pl.ds(0, nrows)], sem_in).wait()
        if c + 1 < n_chunks:
            noff, nn = _PACK_SPANS[c + 1]
            pltpu.async_copy(
                table_hbm.at[pl.ds(rbase + noff, nn)],
                in_v.at[1 - p, pl.ds(0, nn)], sem_in)
        if c >= 2:
            poff, pn = _PACK_SPANS[c - 2]
            pltpu.make_async_copy(
                out_v.at[p, pl.ds(0, pn)],
                out_hbm.at[pl.ds(rbase + poff, pn)], sem_out).wait()
        pack_chunk(p, nrows)
        pltpu.async_copy(
            out_v.at[p, pl.ds(0, nrows)],
            out_hbm.at[pl.ds(rbase + off, nrows)], sem_out)
    for c in (n_chunks - 2, n_chunks - 1):
        off, nrows = _PACK_SPANS[c]
        pltpu.make_async_copy(
            out_v.at[c % 2, pl.ds(0, nrows)],
            out_hbm.at[pl.ds(rbase + off, nrows)], sem_out).wait()


def kernel(input_batch, embedding_table, W1, b1, W2, b2):
    idx = input_batch.astype(jnp.int32).reshape(BATCH * SEQ_LEN)
    packed = _sc_pack_table(embedding_table)
    sums = _sc_gather_sum(idx, packed)
    w2p = jnp.pad(W2, ((0, 0), (0, EMBED_DIM - NUM_CLASSES)))
    b2p = jnp.pad(
        b2.reshape(1, NUM_CLASSES),
        ((0, 0), (0, EMBED_DIM - NUM_CLASSES)),
        constant_values=-1e30,
    )
    return _tc_mlp(sums, W1, b1.reshape(1, HIDDEN_DIM), w2p, b2p)
